# baseline (speedup 1.0000x reference)
import functools

import jax
import jax.numpy as jnp
from jax import lax
from jax.experimental import pallas as pl
from jax.experimental.pallas import tpu as pltpu
from jax.experimental.pallas import tpu_sc as plsc

VOCAB = 30522
DIM = 128


@functools.lru_cache(maxsize=None)
def _make_gather(B: int, D: int, B_out: int):
    info = plsc.get_sparse_core_info()
    NC, NS = info.num_cores, info.num_subcores
    NW = NC * NS
    assert B % NW == 0
    b_per_w = B // NW
    nchunks = 16
    chunk = b_per_w // nchunks
    assert b_per_w % nchunks == 0 and chunk % 8 == 0

    mesh = plsc.VectorSubcoreMesh(core_axis_name="c", subcore_axis_name="s")

    @functools.partial(
        pl.kernel,
        mesh=mesh,
        out_type=jax.ShapeDtypeStruct((B_out, D), jnp.float32),
        scratch_types=[
            pltpu.VMEM((b_per_w,), jnp.int32),
            pltpu.VMEM((2, chunk, D), jnp.float32),
            pltpu.SemaphoreType.DMA,
            pltpu.SemaphoreType.DMA,
            pltpu.SemaphoreType.DMA,
            pltpu.SemaphoreType.DMA,
        ],
    )
    def k(idx_hbm, table_hbm, out_hbm, idx_v, rows_v, g0, g1, w0, w1):
        wid = lax.axis_index("s") * NC + lax.axis_index("c")
        base = wid * b_per_w
        gsem = (g0, g1)
        wsem = (w0, w1)
        pltpu.sync_copy(idx_hbm.at[pl.ds(base, b_per_w)], idx_v)

        gathers = [None, None]
        writes = [None, None]
        gathers[0] = pltpu.async_copy(
            table_hbm.at[idx_v.at[pl.ds(0, chunk)]], rows_v.at[0], gsem[0]
        )
        for g in range(nchunks):
            b = g % 2
            gathers[b].wait()
            if writes[1 - b] is not None:
                writes[1 - b].wait()
            if g + 1 < nchunks:
                gathers[1 - b] = pltpu.async_copy(
                    table_hbm.at[idx_v.at[pl.ds((g + 1) * chunk, chunk)]],
                    rows_v.at[1 - b],
                    gsem[1 - b],
                )
            writes[b] = pltpu.async_copy(
                rows_v.at[b], out_hbm.at[pl.ds(base + g * chunk, chunk)], wsem[b]
            )
        writes[(nchunks - 1) % 2].wait()

    return k


def kernel(x, table):
    nb, ns = x.shape
    ns_tc = 5
    ns_sc = ns - ns_tc
    xt = x.T.astype(jnp.int32)
    idx_sc = xt[:ns_sc].reshape(ns_sc * nb)
    out_sc = _make_gather(ns_sc * nb, DIM, ns * nb)(idx_sc, table)
    out3 = out_sc.reshape(ns, nb, DIM)
    tc_part = jnp.take(table, xt[ns_sc:].reshape(ns_tc * nb), axis=0)
    full = jax.lax.dynamic_update_slice(
        out3, tc_part.reshape(ns_tc, nb, DIM), (ns_sc, 0, 0)
    )
    return full.transpose(1, 0, 2)


# speedup vs baseline: 1.0984x; 1.0984x over previous
import functools

import jax
import jax.numpy as jnp
from jax import lax
from jax.experimental import pallas as pl
from jax.experimental.pallas import tpu as pltpu
from jax.experimental.pallas import tpu_sc as plsc

VOCAB = 30522
DIM = 128


@functools.lru_cache(maxsize=None)
def _make_gather(B: int, D: int):
    info = plsc.get_sparse_core_info()
    NC, NS = info.num_cores, info.num_subcores
    NW = NC * NS
    assert B % NW == 0
    b_per_w = B // NW
    chunk = 400
    nchunks = b_per_w // chunk
    assert b_per_w % chunk == 0 and chunk % 8 == 0

    mesh = plsc.VectorSubcoreMesh(core_axis_name="c", subcore_axis_name="s")

    @functools.partial(
        pl.kernel,
        mesh=mesh,
        out_type=jax.ShapeDtypeStruct((B, D), jnp.float32),
        scratch_types=[
            pltpu.VMEM((b_per_w,), jnp.int32),
            pltpu.VMEM((chunk, D), jnp.float32),
            pltpu.SemaphoreType.DMA,
        ],
    )
    def k(idx_hbm, table_hbm, out_hbm, idx_v, rows_v, sem):
        wid = lax.axis_index("s") * NC + lax.axis_index("c")
        base = wid * b_per_w
        pltpu.sync_copy(idx_hbm.at[pl.ds(base, b_per_w)], idx_v)

        def body(g, carry):
            off = pl.multiple_of(g * chunk, 8)
            pltpu.async_copy(
                table_hbm.at[idx_v.at[pl.ds(off, chunk)]], rows_v, sem
            ).wait()
            pltpu.sync_copy(rows_v, out_hbm.at[pl.ds(base + off, chunk)])
            return carry

        lax.fori_loop(0, nchunks, body, 0)

    return k


def kernel(x, table):
    nb, ns = x.shape
    B = nb * ns
    idx = x.T.reshape(B).astype(jnp.int32)
    out = _make_gather(B, DIM)(idx, table)
    return out.reshape(ns, nb, DIM).transpose(1, 0, 2)


# speedup vs baseline: 1.1612x; 1.0572x over previous
import functools

import jax
import jax.numpy as jnp
from jax import lax
from jax.experimental import pallas as pl
from jax.experimental.pallas import tpu as pltpu
from jax.experimental.pallas import tpu_sc as plsc

VOCAB = 30522
DIM = 128


@functools.lru_cache(maxsize=None)
def _make_gather(B: int, D: int):
    info = plsc.get_sparse_core_info()
    NC, NS = info.num_cores, info.num_subcores
    NW = NC * NS
    assert B % NW == 0
    b_per_w = B // NW
    chunk = 800
    nchunks = b_per_w // chunk
    assert b_per_w % chunk == 0 and chunk % 8 == 0

    mesh = plsc.VectorSubcoreMesh(core_axis_name="c", subcore_axis_name="s")

    @functools.partial(
        pl.kernel,
        mesh=mesh,
        out_type=jax.ShapeDtypeStruct((B, D), jnp.float32),
        scratch_types=[
            pltpu.VMEM((b_per_w,), jnp.int32),
            pltpu.VMEM((chunk, D), jnp.float32),
            pltpu.SemaphoreType.DMA,
        ],
    )
    def k(idx_hbm, table_hbm, out_hbm, idx_v, rows_v, sem):
        wid = lax.axis_index("s") * NC + lax.axis_index("c")
        base = wid * b_per_w
        pltpu.sync_copy(idx_hbm.at[pl.ds(base, b_per_w)], idx_v)

        for g in range(nchunks):
            pltpu.async_copy(
                table_hbm.at[idx_v.at[pl.ds(g * chunk, chunk)]], rows_v, sem
            ).wait()
            pltpu.sync_copy(rows_v, out_hbm.at[pl.ds(base + g * chunk, chunk)])

    return k


def kernel(x, table):
    nb, ns = x.shape
    B = nb * ns
    idx = x.T.reshape(B).astype(jnp.int32)
    out = _make_gather(B, DIM)(idx, table)
    return out.reshape(ns, nb, DIM).transpose(1, 0, 2)


# speedup vs baseline: 1.2012x; 1.0345x over previous
import functools

import jax
import jax.numpy as jnp
from jax import lax
from jax.experimental import pallas as pl
from jax.experimental.pallas import tpu as pltpu
from jax.experimental.pallas import tpu_sc as plsc

VOCAB = 30522
DIM = 128


@functools.lru_cache(maxsize=None)
def _make_gather(B: int, D: int):
    info = plsc.get_sparse_core_info()
    NC, NS = info.num_cores, info.num_subcores
    NW = NC * NS
    assert B % NW == 0
    b_per_w = B // NW
    chunk = 320
    nbuf = 3
    nchunks = b_per_w // chunk
    assert b_per_w % chunk == 0 and chunk % 8 == 0

    mesh = plsc.VectorSubcoreMesh(core_axis_name="c", subcore_axis_name="s")

    @functools.partial(
        pl.kernel,
        mesh=mesh,
        out_type=jax.ShapeDtypeStruct((B, D), jnp.float32),
        scratch_types=[
            pltpu.VMEM((b_per_w,), jnp.int32),
            pltpu.VMEM((3, chunk, D), jnp.float32),
            pltpu.SemaphoreType.DMA,
            pltpu.SemaphoreType.DMA,
            pltpu.SemaphoreType.DMA,
            pltpu.SemaphoreType.DMA,
            pltpu.SemaphoreType.DMA,
            pltpu.SemaphoreType.DMA,
        ],
    )
    def k(idx_hbm, table_hbm, out_hbm, idx_v, rows_v, g0, g1, g2, w0, w1, w2):
        wid = lax.axis_index("s") * NC + lax.axis_index("c")
        base = wid * b_per_w
        gsem = (g0, g1, g2)
        wsem = (w0, w1, w2)
        pltpu.sync_copy(idx_hbm.at[pl.ds(base, b_per_w)], idx_v)

        def gather(g, b):
            return pltpu.async_copy(
                table_hbm.at[idx_v.at[pl.ds(g * chunk, chunk)]],
                rows_v.at[b],
                gsem[b],
            )

        gathers = [None] * nbuf
        writes = [None] * nbuf
        gathers[0] = gather(0, 0)
        gathers[1] = gather(1, 1)
        for g in range(nchunks):
            b = g % nbuf
            bn = (g + 2) % nbuf
            gathers[b].wait()
            if writes[bn] is not None:
                writes[bn].wait()
            if g + 2 < nchunks:
                gathers[bn] = gather(g + 2, bn)
            writes[b] = pltpu.async_copy(
                rows_v.at[b], out_hbm.at[pl.ds(base + g * chunk, chunk)], wsem[b]
            )
        writes[(nchunks - 1) % nbuf].wait()

    return k


def kernel(x, table):
    nb, ns = x.shape
    B = nb * ns
    idx = x.T.reshape(B).astype(jnp.int32)
    out = _make_gather(B, DIM)(idx, table)
    return out.reshape(ns, nb, DIM).transpose(1, 0, 2)
